# baseline (speedup 1.0000x reference)
import functools
import math

import jax
import jax.numpy as jnp
from jax import lax
from jax.experimental import pallas as pl
from jax.experimental.pallas import tpu as pltpu
from jax.experimental.pallas import tpu_sc as plsc

_VOCAB = 100000
_D = 512
_B = 1024
_S = 50
_SCALE = math.sqrt(_D)
_NW = 32
_N = _B * _S
_PER_W = _N // _NW
_C = 80
_STEPS = _PER_W // _C


def _pe_block():
    position = jnp.arange(0, _S, dtype=jnp.float32)[:, None]
    div_term = jnp.exp(
        jnp.arange(0, _D, 2, dtype=jnp.float32) * -(math.log(10000.0) / _D)
    )
    angles = position * div_term
    pe = jnp.zeros((_S, _D), dtype=jnp.float32)
    pe = pe.at[:, 0::2].set(jnp.sin(angles))
    pe = pe.at[:, 1::2].set(jnp.cos(angles))
    return pe


@functools.partial(
    pl.kernel,
    mesh=plsc.VectorSubcoreMesh(core_axis_name="c", subcore_axis_name="s"),
    out_type=jax.ShapeDtypeStruct((_N, _D), jnp.float32),
    scratch_types=[
        pltpu.VMEM((_C,), jnp.int32),
        pltpu.VMEM((_C, _D), jnp.float32),
        pltpu.VMEM((_S, _D), jnp.float32),
        pltpu.SemaphoreType.DMA,
    ],
)
def _emb(table_hbm, idx_hbm, pe_hbm, out_hbm, idx_v, rows_v, pe_v, sem):
    wid = lax.axis_index("s") * 2 + lax.axis_index("c")
    pltpu.sync_copy(pe_hbm, pe_v)

    def step(g, carry):
        base = wid * _PER_W + g * _C
        pltpu.sync_copy(idx_hbm.at[pl.ds(base, _C)], idx_v)
        pltpu.async_copy(table_hbm.at[idx_v], rows_v, sem).wait()
        off = lax.rem(g * _C, _S)

        def row(i, c2):
            r = lax.rem(off + i, _S)
            for v in range(_D // 16):
                sl = pl.ds(v * 16, 16)
                rows_v[i, sl] = rows_v[i, sl] * _SCALE + pe_v[r, sl]
            return c2

        lax.fori_loop(0, _C, row, 0)
        pltpu.sync_copy(rows_v, out_hbm.at[pl.ds(base, _C)])
        return carry

    lax.fori_loop(0, _STEPS, step, 0)


def kernel(x, table):
    idx = x.astype(jnp.int32).reshape(_N)
    pe = _pe_block()
    out = _emb(table, idx, pe)
    return out.reshape(_B, _S, _D)


# speedup vs baseline: 1.1786x; 1.1786x over previous
import functools
import math

import jax
import jax.numpy as jnp
from jax import lax
from jax.experimental import pallas as pl
from jax.experimental.pallas import tpu as pltpu
from jax.experimental.pallas import tpu_sc as plsc

_VOCAB = 100000
_D = 512
_B = 1024
_S = 50
_SCALE = math.sqrt(_D)
_NW = 32
_N = _B * _S
_PER_W = _N // _NW
_C = 40
_STEPS = _PER_W // _C
_NBUF = 4


def _pe_block():
    position = jnp.arange(0, _S, dtype=jnp.float32)[:, None]
    div_term = jnp.exp(
        jnp.arange(0, _D, 2, dtype=jnp.float32) * -(math.log(10000.0) / _D)
    )
    angles = position * div_term
    pe = jnp.zeros((_S, _D), dtype=jnp.float32)
    pe = pe.at[:, 0::2].set(jnp.sin(angles))
    pe = pe.at[:, 1::2].set(jnp.cos(angles))
    return pe


@functools.partial(
    pl.kernel,
    mesh=plsc.VectorSubcoreMesh(core_axis_name="c", subcore_axis_name="s"),
    out_type=jax.ShapeDtypeStruct((_N, _D), jnp.float32),
    scratch_types=[
        pltpu.VMEM((_PER_W,), jnp.int32),
        pltpu.VMEM((_C, _D), jnp.float32),
        pltpu.VMEM((_C, _D), jnp.float32),
        pltpu.VMEM((_C, _D), jnp.float32),
        pltpu.VMEM((_C, _D), jnp.float32),
        pltpu.VMEM((_S, _D), jnp.float32),
        pltpu.SemaphoreType.DMA,
        pltpu.SemaphoreType.DMA,
        pltpu.SemaphoreType.DMA,
        pltpu.SemaphoreType.DMA,
        pltpu.SemaphoreType.DMA,
        pltpu.SemaphoreType.DMA,
        pltpu.SemaphoreType.DMA,
        pltpu.SemaphoreType.DMA,
    ],
)
def _emb(table_hbm, idx_hbm, pe_hbm, out_hbm, idx_v, r0, r1, r2, r3,
         pe_v, g0, g1, g2, g3, w0, w1, w2, w3):
    wid = lax.axis_index("s") * 2 + lax.axis_index("c")
    wbase = wid * _PER_W
    pltpu.sync_copy(pe_hbm, pe_v)
    pltpu.sync_copy(idx_hbm.at[pl.ds(wbase, _PER_W)], idx_v)

    rows = (r0, r1, r2, r3)
    gsem = (g0, g1, g2, g3)
    wsem = (w0, w1, w2, w3)

    def start_gather(g, b):
        off = pl.multiple_of(g * _C, 8)
        pltpu.async_copy(table_hbm.at[idx_v.at[pl.ds(off, _C)]],
                         rows[b], gsem[b])

    def wait_gather(b):
        pltpu.make_async_copy(table_hbm.at[pl.ds(0, _C)], rows[b],
                              gsem[b]).wait()

    def start_write(g, b):
        pltpu.async_copy(rows[b], out_hbm.at[pl.ds(wbase + g * _C, _C)],
                         wsem[b])

    def wait_write(b):
        pltpu.make_async_copy(rows[b], out_hbm.at[pl.ds(wbase, _C)],
                              wsem[b]).wait()

    def compute(g, b):
        off = lax.rem(g * _C, _S)

        def row(i, c2):
            r = lax.rem(off + i, _S)
            for v in range(_D // 16):
                sl = pl.ds(v * 16, 16)
                rows[b][i, sl] = rows[b][i, sl] * _SCALE + pe_v[r, sl]
            return c2

        lax.fori_loop(0, _C, row, 0)

    start_gather(0, 0)
    start_gather(1, 1)

    def body(t, carry):
        for k in range(_NBUF):
            g = t * _NBUF + k
            wait_gather(k)

            @pl.when(g + 2 < _STEPS)
            def _issue_next():
                nxt = (k + 2) % _NBUF
                if k >= 2:
                    wait_write(nxt)
                    start_gather(g + 2, nxt)
                else:
                    @pl.when(t >= 1)
                    def _w():
                        wait_write(nxt)

                    start_gather(g + 2, nxt)

            compute(g, k)
            start_write(g, k)
        return carry

    lax.fori_loop(0, _STEPS // _NBUF, body, 0)
    for k in range(_NBUF):
        wait_write(k)


def kernel(x, table):
    idx = x.astype(jnp.int32).reshape(_N)
    pe = _pe_block()
    out = _emb(table, idx, pe)
    return out.reshape(_B, _S, _D)


# speedup vs baseline: 5.2728x; 4.4738x over previous
import functools
import math

import jax
import jax.numpy as jnp
from jax import lax
from jax.experimental import pallas as pl
from jax.experimental.pallas import tpu as pltpu
from jax.experimental.pallas import tpu_sc as plsc

_VOCAB = 100000
_D = 512
_B = 1024
_S = 50
_SCALE = math.sqrt(_D)
_NW = 32
_N = _B * _S
_C = _B // _NW
_NBUF = 5
_NV = _D // 16


def _pe_block():
    position = jnp.arange(0, _S, dtype=jnp.float32)[:, None]
    div_term = jnp.exp(
        jnp.arange(0, _D, 2, dtype=jnp.float32) * -(math.log(10000.0) / _D)
    )
    angles = position * div_term
    return jnp.stack(
        [jnp.sin(angles), jnp.cos(angles)], axis=-1
    ).reshape(_S, _D)


@functools.partial(
    pl.kernel,
    mesh=plsc.VectorSubcoreMesh(core_axis_name="c", subcore_axis_name="s"),
    out_type=jax.ShapeDtypeStruct((_N, _D), jnp.float32),
    scratch_types=[
        pltpu.VMEM((_S * _C,), jnp.int32),
        pltpu.VMEM((_C, _D), jnp.float32),
        pltpu.VMEM((_C, _D), jnp.float32),
        pltpu.VMEM((_C, _D), jnp.float32),
        pltpu.VMEM((_C, _D), jnp.float32),
        pltpu.VMEM((_C, _D), jnp.float32),
        pltpu.VMEM((_S, _D), jnp.float32),
        pltpu.SemaphoreType.DMA,
        pltpu.SemaphoreType.DMA,
        pltpu.SemaphoreType.DMA,
        pltpu.SemaphoreType.DMA,
        pltpu.SemaphoreType.DMA,
        pltpu.SemaphoreType.DMA,
        pltpu.SemaphoreType.DMA,
        pltpu.SemaphoreType.DMA,
        pltpu.SemaphoreType.DMA,
        pltpu.SemaphoreType.DMA,
    ],
)
def _emb(table_hbm, idx_hbm, pe_hbm, out_hbm, idx_v, r0, r1, r2, r3, r4,
         pe_v, g0, g1, g2, g3, g4, w0, w1, w2, w3, w4):
    wid = lax.axis_index("s") * 2 + lax.axis_index("c")
    col0 = wid * _C
    pltpu.sync_copy(pe_hbm, pe_v)
    pltpu.sync_copy(idx_hbm.at[pl.ds(wid * _S * _C, _S * _C)], idx_v)

    rows = (r0, r1, r2, r3, r4)
    gsem = (g0, g1, g2, g3, g4)
    wsem = (w0, w1, w2, w3, w4)

    def start_gather(g, b):
        off = pl.multiple_of(g * _C, 8)
        pltpu.async_copy(table_hbm.at[idx_v.at[pl.ds(off, _C)]],
                         rows[b], gsem[b])

    def wait_gather(b):
        pltpu.make_async_copy(table_hbm.at[pl.ds(0, _C)], rows[b],
                              gsem[b]).wait()

    def start_write(g, b):
        pltpu.async_copy(rows[b], out_hbm.at[pl.ds(g * _B + col0, _C)],
                         wsem[b])

    def wait_write(b):
        pltpu.make_async_copy(rows[b], out_hbm.at[pl.ds(col0, _C)],
                              wsem[b]).wait()

    def compute(g, b):
        pe_row = [pe_v[g, pl.ds(v * 16, 16)] for v in range(_NV)]

        def row(i, c2):
            for v in range(_NV):
                sl = pl.ds(v * 16, 16)
                rows[b][i, sl] = rows[b][i, sl] * _SCALE + pe_row[v]
            return c2

        lax.fori_loop(0, _C, row, 0)

    start_gather(0, 0)
    start_gather(1, 1)

    def body(t, carry):
        for k in range(_NBUF):
            g = t * _NBUF + k
            wait_gather(k)

            @pl.when(g + 2 < _S)
            def _issue_next():
                nxt = (k + 2) % _NBUF
                if k >= 3:
                    wait_write(nxt)
                    start_gather(g + 2, nxt)
                else:
                    @pl.when(t >= 1)
                    def _w():
                        wait_write(nxt)

                    start_gather(g + 2, nxt)

            compute(g, k)
            start_write(g, k)
        return carry

    lax.fori_loop(0, _S // _NBUF, body, 0)
    for k in range(_NBUF):
        wait_write(k)


def kernel(x, table):
    idx = (
        x.astype(jnp.int32).T.reshape(_S, _NW, _C)
        .transpose(1, 0, 2).reshape(_N)
    )
    pe = _pe_block()
    out = _emb(table, idx, pe)
    return out.reshape(_S, _B, _D).transpose(1, 0, 2)


# speedup vs baseline: 5.3158x; 1.0082x over previous
import functools
import math

import jax
import jax.numpy as jnp
import numpy as np
from jax import lax
from jax.experimental import pallas as pl
from jax.experimental.pallas import tpu as pltpu
from jax.experimental.pallas import tpu_sc as plsc

_VOCAB = 100000
_D = 512
_B = 1024
_S = 50
_SCALE = math.sqrt(_D)
_NW = 32
_N = _B * _S
_C = _B // _NW
_NBUF = 5
_NV = _D // 16


def _pe_block():
    position = np.arange(0, _S, dtype=np.float32)[:, None]
    div_term = np.exp(
        np.arange(0, _D, 2, dtype=np.float32)
        * np.float32(-(math.log(10000.0) / _D))
    )
    angles = position * div_term
    pe = np.stack([np.sin(angles), np.cos(angles)], axis=-1)
    return jnp.asarray(pe.reshape(_S, _D), dtype=jnp.float32)


@functools.partial(
    pl.kernel,
    mesh=plsc.VectorSubcoreMesh(core_axis_name="c", subcore_axis_name="s"),
    out_type=jax.ShapeDtypeStruct((_N, _D), jnp.float32),
    scratch_types=[
        pltpu.VMEM((_S * _C,), jnp.int32),
        pltpu.VMEM((_C, _D), jnp.float32),
        pltpu.VMEM((_C, _D), jnp.float32),
        pltpu.VMEM((_C, _D), jnp.float32),
        pltpu.VMEM((_C, _D), jnp.float32),
        pltpu.VMEM((_C, _D), jnp.float32),
        pltpu.VMEM((_S, _D), jnp.float32),
        pltpu.SemaphoreType.DMA,
        pltpu.SemaphoreType.DMA,
        pltpu.SemaphoreType.DMA,
        pltpu.SemaphoreType.DMA,
        pltpu.SemaphoreType.DMA,
        pltpu.SemaphoreType.DMA,
        pltpu.SemaphoreType.DMA,
        pltpu.SemaphoreType.DMA,
        pltpu.SemaphoreType.DMA,
        pltpu.SemaphoreType.DMA,
    ],
)
def _emb(table_hbm, idx_hbm, pe_hbm, out_hbm, idx_v, r0, r1, r2, r3, r4,
         pe_v, g0, g1, g2, g3, g4, w0, w1, w2, w3, w4):
    wid = lax.axis_index("s") * 2 + lax.axis_index("c")
    col0 = wid * _C
    pltpu.sync_copy(idx_hbm.at[pl.ds(wid * _S * _C, _S * _C)], idx_v)

    rows = (r0, r1, r2, r3, r4)
    gsem = (g0, g1, g2, g3, g4)
    wsem = (w0, w1, w2, w3, w4)

    def start_gather(g, b):
        off = pl.multiple_of(g * _C, 8)
        pltpu.async_copy(table_hbm.at[idx_v.at[pl.ds(off, _C)]],
                         rows[b], gsem[b])

    def wait_gather(b):
        pltpu.make_async_copy(table_hbm.at[pl.ds(0, _C)], rows[b],
                              gsem[b]).wait()

    def start_write(g, b):
        pltpu.async_copy(rows[b], out_hbm.at[pl.ds(g * _B + col0, _C)],
                         wsem[b])

    def wait_write(b):
        pltpu.make_async_copy(rows[b], out_hbm.at[pl.ds(col0, _C)],
                              wsem[b]).wait()

    def compute(g, b):
        pe_row = [pe_v[g, pl.ds(v * 16, 16)] for v in range(_NV)]

        def row(i, c2):
            for v in range(_NV):
                sl = pl.ds(v * 16, 16)
                rows[b][i, sl] = rows[b][i, sl] * _SCALE + pe_row[v]
            return c2

        lax.fori_loop(0, _C, row, 0)

    start_gather(0, 0)
    start_gather(1, 1)
    pltpu.sync_copy(pe_hbm, pe_v)

    def body(t, carry):
        for k in range(_NBUF):
            g = t * _NBUF + k
            wait_gather(k)

            @pl.when(g + 2 < _S)
            def _issue_next():
                nxt = (k + 2) % _NBUF
                if k >= 3:
                    wait_write(nxt)
                    start_gather(g + 2, nxt)
                else:
                    @pl.when(t >= 1)
                    def _w():
                        wait_write(nxt)

                    start_gather(g + 2, nxt)

            compute(g, k)
            start_write(g, k)
        return carry

    lax.fori_loop(0, _S // _NBUF, body, 0)
    for k in range(_NBUF):
        wait_write(k)


def kernel(x, table):
    idx = (
        x.astype(jnp.int32).T.reshape(_S, _NW, _C)
        .transpose(1, 0, 2).reshape(_N)
    )
    pe = _pe_block()
    out = _emb(table, idx, pe)
    return out.reshape(_S, _B, _D).transpose(1, 0, 2)


# speedup vs baseline: 5.3392x; 1.0044x over previous
import functools
import math

import jax
import jax.numpy as jnp
import numpy as np
from jax import lax
from jax.experimental import pallas as pl
from jax.experimental.pallas import tpu as pltpu
from jax.experimental.pallas import tpu_sc as plsc

_VOCAB = 100000
_D = 512
_B = 1024
_S = 50
_SCALE = math.sqrt(_D)
_NW = 32
_N = _B * _S
_C = _B // _NW
_NBUF = 5
_NV = _D // 16


def _pe_block():
    position = np.arange(0, _S, dtype=np.float32)[:, None]
    div_term = np.exp(
        np.arange(0, _D, 2, dtype=np.float32)
        * np.float32(-(math.log(10000.0) / _D))
    )
    angles = position * div_term
    pe = np.stack([np.sin(angles), np.cos(angles)], axis=-1)
    return jnp.asarray(pe.reshape(_S, _D), dtype=jnp.float32)


@functools.partial(
    pl.kernel,
    mesh=plsc.VectorSubcoreMesh(core_axis_name="c", subcore_axis_name="s"),
    out_type=jax.ShapeDtypeStruct((_N, _D), jnp.float32),
    scratch_types=[
        pltpu.VMEM((_S * _C,), jnp.int32),
        pltpu.VMEM((_C, _D), jnp.float32),
        pltpu.VMEM((_C, _D), jnp.float32),
        pltpu.VMEM((_C, _D), jnp.float32),
        pltpu.VMEM((_C, _D), jnp.float32),
        pltpu.VMEM((_C, _D), jnp.float32),
        pltpu.VMEM((_S, _D), jnp.float32),
        pltpu.SemaphoreType.DMA,
        pltpu.SemaphoreType.DMA,
        pltpu.SemaphoreType.DMA,
        pltpu.SemaphoreType.DMA,
        pltpu.SemaphoreType.DMA,
        pltpu.SemaphoreType.DMA,
        pltpu.SemaphoreType.DMA,
        pltpu.SemaphoreType.DMA,
        pltpu.SemaphoreType.DMA,
        pltpu.SemaphoreType.DMA,
    ],
)
def _emb(table_hbm, idx_hbm, pe_hbm, out_hbm, idx_v, r0, r1, r2, r3, r4,
         pe_v, g0, g1, g2, g3, g4, w0, w1, w2, w3, w4):
    wid = lax.axis_index("s") * 2 + lax.axis_index("c")
    col0 = wid * _C
    pltpu.sync_copy(idx_hbm.at[pl.ds(wid * _S * _C, _S * _C)], idx_v)

    rows = (r0, r1, r2, r3, r4)
    gsem = (g0, g1, g2, g3, g4)
    wsem = (w0, w1, w2, w3, w4)

    def start_gather(g, b):
        off = pl.multiple_of(g * _C, 8)
        pltpu.async_copy(table_hbm.at[idx_v.at[pl.ds(off, _C)]],
                         rows[b], gsem[b])

    def wait_gather(b):
        pltpu.make_async_copy(table_hbm.at[pl.ds(0, _C)], rows[b],
                              gsem[b]).wait()

    def start_write(g, b):
        pltpu.async_copy(rows[b], out_hbm.at[pl.ds(g * _B + col0, _C)],
                         wsem[b])

    def wait_write(b):
        pltpu.make_async_copy(rows[b], out_hbm.at[pl.ds(col0, _C)],
                              wsem[b]).wait()

    def compute(g, b):
        pe_row = [pe_v[g, pl.ds(v * 16, 16)] for v in range(_NV)]

        def row(i, c2):
            for v in range(_NV):
                sl = pl.ds(v * 16, 16)
                rows[b][i, sl] = rows[b][i, sl] * _SCALE + pe_row[v]
            return c2

        lax.fori_loop(0, _C, row, 0)

    start_gather(0, 0)
    start_gather(1, 1)
    start_gather(2, 2)
    pltpu.sync_copy(pe_hbm, pe_v)

    def body(t, carry):
        for k in range(_NBUF):
            g = t * _NBUF + k
            wait_gather(k)

            @pl.when(g + 3 < _S)
            def _issue_next():
                nxt = (k + 3) % _NBUF
                if k >= 2:
                    wait_write(nxt)
                    start_gather(g + 3, nxt)
                else:
                    @pl.when(t >= 1)
                    def _w():
                        wait_write(nxt)

                    start_gather(g + 3, nxt)

            compute(g, k)
            start_write(g, k)
        return carry

    lax.fori_loop(0, _S // _NBUF, body, 0)
    for k in range(_NBUF):
        wait_write(k)


def kernel(x, table):
    idx = (
        x.astype(jnp.int32).T.reshape(_S, _NW, _C)
        .transpose(1, 0, 2).reshape(_N)
    )
    pe = _pe_block()
    out = _emb(table, idx, pe)
    return out.reshape(_S, _B, _D).transpose(1, 0, 2)
